# baseline (speedup 1.0000x reference)
import functools

import jax
import jax.numpy as jnp
from jax.experimental import pallas as pl

N = 4096
FEAT = 128
HID = 16
NH = 9
NC = 27
BLK = 512
KCH = N // 4


def _support_body(x_ref, w_ref, out_ref):
    out_ref[0] = jnp.dot(x_ref[0], w_ref[0], preferred_element_type=jnp.float32)


def _main_body(adj0_ref, adj1_ref, adj2_ref, adj3_ref, sup_ref, b_gc_ref,
               w_mlp_ref, b_mlp_ref, w_cls_ref, b_cls_ref, out_ref):
    i = pl.program_id(1)
    sup = sup_ref[0].astype(jnp.bfloat16)
    h = jnp.zeros((BLK, HID), jnp.float32)
    for j, a_ref in enumerate((adj0_ref, adj1_ref, adj2_ref, adj3_ref)):
        adj = a_ref[0].astype(jnp.bfloat16)
        adj = jnp.where(jnp.isnan(adj), jnp.bfloat16(0.0), adj)
        h = h + jnp.dot(adj, sup[j * KCH:(j + 1) * KCH, :],
                        preferred_element_type=jnp.float32)
    h = h + b_gc_ref[i]
    t = jnp.tanh(jnp.dot(h, w_mlp_ref[i], preferred_element_type=jnp.float32)
                 + b_mlp_ref[i])
    w_cls_i = w_cls_ref[pl.ds(i * NH, NH), :]
    contrib = jnp.dot(t, w_cls_i, preferred_element_type=jnp.float32)

    @pl.when(i == 0)
    def _():
        out_ref[...] = contrib + b_cls_ref[0]

    @pl.when(i != 0)
    def _():
        out_ref[...] += contrib


@jax.jit
def kernel(x, adjs, W_gc, b_gc, W_mlp, b_mlp, W_cls, b_cls):
    support = pl.pallas_call(
        _support_body,
        grid=(3,),
        in_specs=[
            pl.BlockSpec((1, N, FEAT), lambda i: (i, 0, 0)),
            pl.BlockSpec((1, FEAT, HID), lambda i: (i, 0, 0)),
        ],
        out_specs=pl.BlockSpec((1, N, HID), lambda i: (i, 0, 0)),
        out_shape=jax.ShapeDtypeStruct((3, N, HID), jnp.float32),
    )(x, W_gc)

    nb = N // BLK
    adj_specs = [
        pl.BlockSpec((1, BLK, KCH), functools.partial(
            lambda b, i, jj: (i, b, jj), jj=j))
        for j in range(4)
    ]
    out = pl.pallas_call(
        _main_body,
        grid=(nb, 3),
        in_specs=adj_specs + [
            pl.BlockSpec((1, N, HID), lambda b, i: (i, 0, 0)),
            pl.BlockSpec((3, HID), lambda b, i: (0, 0)),
            pl.BlockSpec((3, HID, NH), lambda b, i: (0, 0, 0)),
            pl.BlockSpec((3, NH), lambda b, i: (0, 0)),
            pl.BlockSpec((3 * NH, NC), lambda b, i: (0, 0)),
            pl.BlockSpec((1, NC), lambda b, i: (0, 0)),
        ],
        out_specs=pl.BlockSpec((BLK, NC), lambda b, i: (b, 0)),
        out_shape=jax.ShapeDtypeStruct((N, NC), jnp.float32),
    )(adjs, adjs, adjs, adjs, support, b_gc, W_mlp, b_mlp, W_cls,
      b_cls.reshape(1, NC))
    return out


# speedup vs baseline: 1.0632x; 1.0632x over previous
import functools

import jax
import jax.numpy as jnp
from jax.experimental import pallas as pl
from jax.experimental.pallas import tpu as pltpu

N = 4096
FEAT = 128
HID = 16
NH = 9
NC = 27
BLK = 1024


def _support_body(x_ref, w_ref, o_ref):
    o_ref[0] = jnp.dot(x_ref[0], w_ref[0], preferred_element_type=jnp.float32)


@jax.jit
def kernel(x, adjs, W_gc, b_gc, W_mlp, b_mlp, W_cls, b_cls):
    support = pl.pallas_call(
        _support_body,
        grid=(3,),
        in_specs=[
            pl.BlockSpec((1, N, FEAT), lambda i: (i, 0, 0)),
            pl.BlockSpec((1, FEAT, HID), lambda i: (i, 0, 0)),
        ],
        out_specs=pl.BlockSpec((1, N, HID), lambda i: (i, 0, 0)),
        out_shape=jax.ShapeDtypeStruct((3, N, HID), jnp.float32),
    )(x, W_gc)

    nb = N // BLK

    def body(adj_ref, sup_ref, b_gc_ref, w_mlp_ref, b_mlp_ref, w_cls_ref,
             b_cls_ref, out_ref):
        i = pl.program_id(1)
        adj = adj_ref[0]
        adj = jnp.where(jnp.isnan(adj), 0.0, adj)
        h = jnp.dot(adj, sup_ref[0], preferred_element_type=jnp.float32)
        h = h + b_gc_ref[i]
        t = jnp.tanh(jnp.dot(h, w_mlp_ref[i],
                             preferred_element_type=jnp.float32)
                     + b_mlp_ref[i])
        w_cls_i = w_cls_ref[pl.ds(i * NH, NH), :]
        contrib = jnp.dot(t, w_cls_i, preferred_element_type=jnp.float32)

        @pl.when(i == 0)
        def _():
            out_ref[...] = contrib + b_cls_ref[0]

        @pl.when(i != 0)
        def _():
            out_ref[...] += contrib

    out = pl.pallas_call(
        body,
        grid=(nb, 3),
        in_specs=[
            pl.BlockSpec((1, BLK, N), lambda b, i: (i, b, 0)),
            pl.BlockSpec((1, N, HID), lambda b, i: (i, 0, 0)),
            pl.BlockSpec((3, HID), lambda b, i: (0, 0)),
            pl.BlockSpec((3, HID, NH), lambda b, i: (0, 0, 0)),
            pl.BlockSpec((3, NH), lambda b, i: (0, 0)),
            pl.BlockSpec((3 * NH, NC), lambda b, i: (0, 0)),
            pl.BlockSpec((1, NC), lambda b, i: (0, 0)),
        ],
        out_specs=pl.BlockSpec((BLK, NC), lambda b, i: (b, 0)),
        out_shape=jax.ShapeDtypeStruct((N, NC), jnp.float32),
    )(adjs, support, b_gc, W_mlp, b_mlp, W_cls, b_cls.reshape(1, NC))
    return out


# speedup vs baseline: 1.2562x; 1.1816x over previous
import jax
import jax.numpy as jnp
from jax.experimental import pallas as pl
from jax.experimental.pallas import tpu as pltpu

N = 4096
FEAT = 128
HID = 16
NH = 9
NC = 27
BLK = 1024


def _body(x_ref, adj_ref, w_gc_ref, b_gc_ref, w_mlp_ref, b_mlp_ref,
          w_cls_ref, b_cls_ref, out_ref, sup_ref):
    b = pl.program_id(0)
    i = pl.program_id(1)

    @pl.when(b == 0)
    def _():
        sup_ref[i] = jnp.dot(x_ref[i], w_gc_ref[i],
                             preferred_element_type=jnp.float32)

    adj = adj_ref[0]
    adj = jnp.where(jnp.isnan(adj), 0.0, adj)
    h = jnp.dot(adj, sup_ref[i], preferred_element_type=jnp.float32)
    h = h + b_gc_ref[i]
    t = jnp.tanh(jnp.dot(h, w_mlp_ref[i], preferred_element_type=jnp.float32)
                 + b_mlp_ref[i])
    w_cls_i = w_cls_ref[pl.ds(i * NH, NH), :]
    contrib = jnp.dot(t, w_cls_i, preferred_element_type=jnp.float32)

    @pl.when(i == 0)
    def _():
        out_ref[...] = contrib + b_cls_ref[0]

    @pl.when(i != 0)
    def _():
        out_ref[...] += contrib


@jax.jit
def kernel(x, adjs, W_gc, b_gc, W_mlp, b_mlp, W_cls, b_cls):
    nb = N // BLK
    out = pl.pallas_call(
        _body,
        grid=(nb, 3),
        in_specs=[
            pl.BlockSpec((3, N, FEAT), lambda b, i: (0, 0, 0)),
            pl.BlockSpec((1, BLK, N), lambda b, i: (i, b, 0)),
            pl.BlockSpec((3, FEAT, HID), lambda b, i: (0, 0, 0)),
            pl.BlockSpec((3, HID), lambda b, i: (0, 0)),
            pl.BlockSpec((3, HID, NH), lambda b, i: (0, 0, 0)),
            pl.BlockSpec((3, NH), lambda b, i: (0, 0)),
            pl.BlockSpec((3 * NH, NC), lambda b, i: (0, 0)),
            pl.BlockSpec((1, NC), lambda b, i: (0, 0)),
        ],
        out_specs=pl.BlockSpec((BLK, NC), lambda b, i: (b, 0)),
        out_shape=jax.ShapeDtypeStruct((N, NC), jnp.float32),
        scratch_shapes=[pltpu.VMEM((3, N, HID), jnp.float32)],
    )(x, adjs, W_gc, b_gc, W_mlp, b_mlp, W_cls, b_cls.reshape(1, NC))
    return out
